# Initial kernel scaffold; baseline (speedup 1.0000x reference)
import jax
import jax.numpy as jnp
from jax.experimental import pallas as pl


def kernel(x, co, pos_table, word_table, Wx, Wh, bx, bh, Wo, bo):
    raise NotImplementedError("write your pallas kernel here")


# speedup vs baseline: 7.1352x; 7.1352x over previous
import functools

import jax
import jax.numpy as jnp
from jax import lax
from jax.experimental import pallas as pl
from jax.experimental.pallas import tpu as pltpu
from jax.experimental.pallas import tpu_sc as plsc

B = 4
BP = 8
T = 2048
E = 256
NW = 32
ROWS = T * BP
GCH = 512
GSUB = 128
TCH = 256
NCH = T // TCH


def _sc_mesh():
    return plsc.VectorSubcoreMesh(core_axis_name="c", subcore_axis_name="s")


def _gather_body(tbl, idxh, out, idx_v, rows_v, sem):
    cid = lax.axis_index("c")
    sid = lax.axis_index("s")
    w = sid * 2 + cid
    for k in range(GCH // GSUB):
        pltpu.sync_copy(idxh.at[w, k], idx_v)
        pltpu.async_copy(tbl.at[idx_v], rows_v, sem).wait()
        pltpu.sync_copy(rows_v, out.at[pl.ds(w * GCH + k * GSUB, GSUB)])


def _word_gather(word_table, idx_sc):
    f = pl.kernel(
        _gather_body,
        out_type=jax.ShapeDtypeStruct((ROWS, E), jnp.float32),
        mesh=_sc_mesh(),
        scratch_types=[
            pltpu.VMEM((GSUB,), jnp.int32),
            pltpu.VMEM((GSUB, E), jnp.float32),
            pltpu.SemaphoreType.DMA,
        ],
    )
    return f(word_table, idx_sc)


def _gx_body(tag, posq, ptab, wxp, wxw, bias, out, pg):
    @pl.when(pl.program_id(0) == 0)
    def _():
        pg[...] = jnp.dot(ptab[...], wxp[...], preferred_element_type=jnp.float32)

    rows = tag.shape[0]
    oh = (posq[...] == lax.broadcasted_iota(jnp.int32, (rows, 64), 1)).astype(
        jnp.float32)
    out[...] = (jnp.dot(tag[...], wxw[...], preferred_element_type=jnp.float32)
                + jnp.dot(oh, pg[...], preferred_element_type=jnp.float32)
                + bias[...])


def _gx(tag, posq, pos_table, wxp, wxw, bias):
    rows = TCH * BP
    return pl.pallas_call(
        _gx_body,
        grid=(NCH,),
        in_specs=[
            pl.BlockSpec((rows, E), lambda i: (i, 0)),
            pl.BlockSpec((rows, 1), lambda i: (i, 0)),
            pl.BlockSpec((64, E), lambda i: (0, 0)),
            pl.BlockSpec((E, 3 * E), lambda i: (0, 0)),
            pl.BlockSpec((E, 3 * E), lambda i: (0, 0)),
            pl.BlockSpec((1, 3 * E), lambda i: (0, 0)),
        ],
        out_specs=pl.BlockSpec((rows, 3 * E), lambda i: (i, 0)),
        out_shape=jax.ShapeDtypeStruct((ROWS, 3 * E), jnp.float32),
        scratch_shapes=[pltpu.VMEM((64, 3 * E), jnp.float32)],
    )(tag, posq, pos_table, wxp, wxw, bias)


def _gru_body(gx, wh, bhn, wo, s_out, h_ref, hs_ref):
    @pl.when(pl.program_id(0) == 0)
    def _():
        h_ref[...] = jnp.zeros((BP, E), jnp.float32)

    whv = wh[...]
    bhnv = bhn[...]

    def step(i, h):
        g = gx[pl.ds(i * BP, BP), :]
        gh = jnp.dot(h, whv, preferred_element_type=jnp.float32)
        r = jax.nn.sigmoid(g[:, 0:E] + gh[:, 0:E])
        z = jax.nn.sigmoid(g[:, E:2 * E] + gh[:, E:2 * E])
        n = jnp.tanh(g[:, 2 * E:3 * E] + r * (gh[:, 2 * E:3 * E] + bhnv))
        h2 = (1.0 - z) * n + z * h
        hs_ref[pl.ds(i * BP, BP), :] = h2
        return h2

    h = lax.fori_loop(0, TCH, step, h_ref[...])
    h_ref[...] = h
    s_out[...] = jnp.sum(hs_ref[...] * wo[...], axis=1, keepdims=True)


def _gru(gxa, wh, bhn, wo_row):
    rows = TCH * BP
    return pl.pallas_call(
        _gru_body,
        grid=(NCH,),
        in_specs=[
            pl.BlockSpec((rows, 3 * E), lambda i: (i, 0)),
            pl.BlockSpec((E, 3 * E), lambda i: (0, 0)),
            pl.BlockSpec((1, E), lambda i: (0, 0)),
            pl.BlockSpec((1, E), lambda i: (0, 0)),
        ],
        out_specs=pl.BlockSpec((rows, 1), lambda i: (i, 0)),
        out_shape=jax.ShapeDtypeStruct((ROWS, 1), jnp.float32),
        scratch_shapes=[
            pltpu.VMEM((BP, E), jnp.float32),
            pltpu.VMEM((rows, E), jnp.float32),
        ],
    )(gxa, wh, bhn, wo_row)


def _mix_body(s_h, co_h, out_h, s_v, co_v, out_v):
    cid = lax.axis_index("c")
    sid = lax.axis_index("s")
    w = sid * 2 + cid
    pltpu.sync_copy(s_h, s_v)
    pltpu.sync_copy(co_h.at[w], co_v)
    nper = (B * T) // NW
    b = w // (T // nper)
    t0 = (w % (T // nper)) * nper
    for j in range(nper // 16):
        cj = co_v[pl.ds(j * 16, 16)]
        ga = plsc.load_gather(s_v, [cj * BP + b])
        tv = lax.broadcasted_iota(jnp.int32, (16,), 0) + (t0 + j * 16)
        gs = plsc.load_gather(s_v, [tv * BP + b])
        out_v[pl.ds(j * 16, 16)] = 0.5 * (gs + ga)
    pltpu.sync_copy(out_v, out_h.at[pl.ds(w * nper, nper)])


def _mix(s_flat, co_rs):
    nper = (B * T) // NW
    f = pl.kernel(
        _mix_body,
        out_type=jax.ShapeDtypeStruct((B * T,), jnp.float32),
        mesh=_sc_mesh(),
        compiler_params=pltpu.CompilerParams(needs_layout_passes=False),
        scratch_types=[
            pltpu.VMEM((ROWS,), jnp.float32),
            pltpu.VMEM((nper,), jnp.int32),
            pltpu.VMEM((nper,), jnp.float32),
        ],
    )
    return f(s_flat, co_rs)


def _sm_body(m, o):
    v = m[...]
    mx = jnp.max(v, axis=1, keepdims=True)
    e = jnp.exp(v - mx)
    o[...] = e / jnp.sum(e, axis=1, keepdims=True)


def _softmax(mix2):
    return pl.pallas_call(
        _sm_body,
        out_shape=jax.ShapeDtypeStruct((B, T), jnp.float32),
    )(mix2)


def kernel(x, co, pos_table, word_table, Wx, Wh, bx, bh, Wo, bo):
    pos_idx = x[:, :, 0]
    word_idx = x[:, :, 1]

    wpad = jnp.zeros((T, BP), jnp.int32).at[:, :B].set(word_idx.T)
    ppad = jnp.zeros((T, BP), jnp.int32).at[:, :B].set(pos_idx.T)
    idx_sc = wpad.reshape(NW, GCH // GSUB, GSUB)

    tag = _word_gather(word_table, idx_sc)

    bias = (bx + jnp.concatenate([bh[:2 * E], jnp.zeros((E,), jnp.float32)]))
    gxa = _gx(tag, ppad.reshape(ROWS, 1), pos_table, Wx[:E], Wx[E:],
              bias.reshape(1, 3 * E))

    s = _gru(gxa, Wh, bh[2 * E:].reshape(1, E), Wo.reshape(1, E))

    mix = _mix(s.reshape(ROWS), co.reshape(NW, (B * T) // NW))

    return _softmax(mix.reshape(B, T))


# speedup vs baseline: 13.0683x; 1.8315x over previous
import jax
import jax.numpy as jnp
from jax import lax
from jax.experimental import pallas as pl
from jax.experimental.pallas import tpu as pltpu
from jax.experimental.pallas import tpu_sc as plsc

B = 4
T = 2048
E = 256
NW = 32
ROWS = T * B
GSUB = 128
GCH = ROWS // NW
NISS = GCH // GSUB
TCH = 256
NCH = T // TCH


def _sc_mesh():
    return plsc.VectorSubcoreMesh(core_axis_name="c", subcore_axis_name="s")


def _gather_body(tbl, idxh, out, idx0, idx1, buf0, buf1, sg0, sg1, so0, so1):
    cid = lax.axis_index("c")
    sid = lax.axis_index("s")
    w = sid * 2 + cid
    base = w * GCH
    pltpu.sync_copy(idxh.at[w, 0], idx0)
    pltpu.sync_copy(idxh.at[w, 1], idx1)
    g0 = pltpu.async_copy(tbl.at[idx0], buf0, sg0)
    g1 = pltpu.async_copy(tbl.at[idx1], buf1, sg1)
    g0.wait()
    o0 = pltpu.async_copy(buf0, out.at[pl.ds(base, GSUB)], so0)
    g1.wait()
    o1 = pltpu.async_copy(buf1, out.at[pl.ds(base + GSUB, GSUB)], so1)
    o0.wait()
    o1.wait()


def _word_gather(word_table, idx_sc):
    f = pl.kernel(
        _gather_body,
        out_type=jax.ShapeDtypeStruct((ROWS, E), jnp.float32),
        mesh=_sc_mesh(),
        scratch_types=[
            pltpu.VMEM((GSUB,), jnp.int32),
            pltpu.VMEM((GSUB,), jnp.int32),
            pltpu.VMEM((GSUB, E), jnp.float32),
            pltpu.VMEM((GSUB, E), jnp.float32),
            pltpu.SemaphoreType.DMA,
            pltpu.SemaphoreType.DMA,
            pltpu.SemaphoreType.DMA,
            pltpu.SemaphoreType.DMA,
        ],
    )
    return f(word_table, idx_sc)


def _gx_body(tag, posq, ptab, wxp, wxw, bias, out, pg):
    @pl.when(pl.program_id(0) == 0)
    def _():
        pg[...] = jnp.dot(ptab[...], wxp[...], preferred_element_type=jnp.float32)

    rows = tag.shape[0]
    oh = (posq[...] == lax.broadcasted_iota(jnp.int32, (rows, 64), 1)).astype(
        jnp.float32)
    out[...] = (jnp.dot(tag[...], wxw[...], preferred_element_type=jnp.float32)
                + jnp.dot(oh, pg[...], preferred_element_type=jnp.float32)
                + bias[...])


def _gx(tag, posq, pos_table, wxp, wxw, bias):
    rows = TCH * B
    return pl.pallas_call(
        _gx_body,
        grid=(NCH,),
        in_specs=[
            pl.BlockSpec((rows, E), lambda i: (i, 0)),
            pl.BlockSpec((rows, 1), lambda i: (i, 0)),
            pl.BlockSpec((64, E), lambda i: (0, 0)),
            pl.BlockSpec((E, 3 * E), lambda i: (0, 0)),
            pl.BlockSpec((E, 3 * E), lambda i: (0, 0)),
            pl.BlockSpec((1, 3 * E), lambda i: (0, 0)),
        ],
        out_specs=pl.BlockSpec((rows, 3 * E), lambda i: (i, 0)),
        out_shape=jax.ShapeDtypeStruct((ROWS, 3 * E), jnp.float32),
        scratch_shapes=[pltpu.VMEM((64, 3 * E), jnp.float32)],
    )(tag, posq, pos_table, wxp, wxw, bias)


def _gru_body(gx, wh, bhn, wo, s_out, h_ref, hs_ref):
    @pl.when(pl.program_id(0) == 0)
    def _():
        h_ref[...] = jnp.zeros((B, E), jnp.float32)

    whv = wh[...]
    bhnv = bhn[...]

    def substep(g, h):
        gh = jnp.dot(h, whv, preferred_element_type=jnp.float32)
        r = jax.nn.sigmoid(g[:, 0:E] + gh[:, 0:E])
        z = jax.nn.sigmoid(g[:, E:2 * E] + gh[:, E:2 * E])
        n = jnp.tanh(g[:, 2 * E:3 * E] + r * (gh[:, 2 * E:3 * E] + bhnv))
        return (1.0 - z) * n + z * h

    def step(i, h):
        g2 = gx[pl.ds(i * 2 * B, 2 * B), :]
        ha = substep(g2[0:B, :], h)
        hb = substep(g2[B:2 * B, :], ha)
        hs_ref[pl.ds(i * 2 * B, 2 * B), :] = jnp.concatenate([ha, hb], axis=0)
        return hb

    h = lax.fori_loop(0, TCH // 2, step, h_ref[...])
    h_ref[...] = h
    s_out[...] = jnp.sum(hs_ref[...] * wo[...], axis=1, keepdims=True)


def _gru(gxa, wh, bhn, wo_row):
    rows = TCH * B
    return pl.pallas_call(
        _gru_body,
        grid=(NCH,),
        in_specs=[
            pl.BlockSpec((rows, 3 * E), lambda i: (i, 0)),
            pl.BlockSpec((E, 3 * E), lambda i: (0, 0)),
            pl.BlockSpec((1, E), lambda i: (0, 0)),
            pl.BlockSpec((1, E), lambda i: (0, 0)),
        ],
        out_specs=pl.BlockSpec((rows, 1), lambda i: (i, 0)),
        out_shape=jax.ShapeDtypeStruct((ROWS, 1), jnp.float32),
        scratch_shapes=[
            pltpu.VMEM((B, E), jnp.float32),
            pltpu.VMEM((rows, E), jnp.float32),
        ],
    )(gxa, wh, bhn, wo_row)


def _mix_body(s_h, co_h, out_h, s_v, co_v, out_v):
    cid = lax.axis_index("c")
    sid = lax.axis_index("s")
    w = sid * 2 + cid
    pltpu.sync_copy(s_h, s_v)
    pltpu.sync_copy(co_h.at[w], co_v)
    nper = (B * T) // NW
    b = w // (T // nper)
    t0 = (w % (T // nper)) * nper
    for j in range(nper // 16):
        cj = co_v[pl.ds(j * 16, 16)]
        ga = plsc.load_gather(s_v, [cj * B + b])
        tv = lax.broadcasted_iota(jnp.int32, (16,), 0) + (t0 + j * 16)
        gs = plsc.load_gather(s_v, [tv * B + b])
        out_v[pl.ds(j * 16, 16)] = 0.5 * (gs + ga)
    pltpu.sync_copy(out_v, out_h.at[pl.ds(w * nper, nper)])


def _mix(s_flat, co_rs):
    nper = (B * T) // NW
    f = pl.kernel(
        _mix_body,
        out_type=jax.ShapeDtypeStruct((B * T,), jnp.float32),
        mesh=_sc_mesh(),
        compiler_params=pltpu.CompilerParams(needs_layout_passes=False),
        scratch_types=[
            pltpu.VMEM((ROWS,), jnp.float32),
            pltpu.VMEM((nper,), jnp.int32),
            pltpu.VMEM((nper,), jnp.float32),
        ],
    )
    return f(s_flat, co_rs)


def _sm_body(m, o):
    v = m[...]
    mx = jnp.max(v, axis=1, keepdims=True)
    e = jnp.exp(v - mx)
    o[...] = e / jnp.sum(e, axis=1, keepdims=True)


def _softmax(mix2):
    return pl.pallas_call(
        _sm_body,
        out_shape=jax.ShapeDtypeStruct((B, T), jnp.float32),
    )(mix2)


def kernel(x, co, pos_table, word_table, Wx, Wh, bx, bh, Wo, bo):
    pos_idx = x[:, :, 0]
    word_idx = x[:, :, 1]

    idx_sc = word_idx.T.reshape(NW, NISS, GSUB)

    tag = _word_gather(word_table, idx_sc)

    bias = (bx + jnp.concatenate([bh[:2 * E], jnp.zeros((E,), jnp.float32)]))
    gxa = _gx(tag, pos_idx.T.reshape(ROWS, 1), pos_table, Wx[:E], Wx[E:],
              bias.reshape(1, 3 * E))

    s = _gru(gxa, Wh, bh[2 * E:].reshape(1, E), Wo.reshape(1, E))

    mix = _mix(s.reshape(ROWS), co.reshape(NW, (B * T) // NW))

    return _softmax(mix.reshape(B, T))
